# baseline (speedup 1.0000x reference)
import functools

import jax
import jax.numpy as jnp
from jax import lax
from jax.experimental import pallas as pl
from jax.experimental.pallas import tpu as pltpu
from jax.experimental.pallas import tpu_sc as plsc

_LANES = 16
_CHUNK = 80


def _make_argmax_part1(b, n, vocab, n_workers):
    bpw = b // n_workers
    full = vocab // _LANES
    tail = vocab - full * _LANES
    row_w = n * _LANES

    def _body(x_hbm, bv_hbm, bi_hbm, slab_v, bv_buf, bi_buf, sem, osem):
        wid = lax.axis_index("s") * 2 + lax.axis_index("c")
        base = wid * bpw
        lane = lax.iota(jnp.int32, _LANES)

        pltpu.async_copy(x_hbm.at[base], slab_v.at[0], sem)

        def slab_step(s, _):
            par = s % 2
            off = par * 1024
            pltpu.make_async_copy(x_hbm.at[base + s], slab_v.at[par], sem).wait()

            @pl.when(s + 1 < bpw)
            def _():
                pltpu.async_copy(x_hbm.at[base + s + 1], slab_v.at[1 - par], sem)

            @pl.when(s >= 2)
            def _():
                pltpu.make_async_copy(
                    bv_buf.at[pl.ds(0, row_w)],
                    bv_hbm.at[pl.ds(base * row_w, row_w)], osem).wait()
                pltpu.make_async_copy(
                    bi_buf.at[pl.ds(0, row_w)],
                    bi_hbm.at[pl.ds(base * row_w, row_w)], osem).wait()

            def row_step(r, _):
                best_v = slab_v[par, r, pl.ds(0, _LANES)]
                best_i = lane
                for j in range(1, full):
                    v = slab_v[par, r, pl.ds(j * _LANES, _LANES)]
                    upd = v > best_v
                    best_v = jnp.where(upd, v, best_v)
                    best_i = jnp.where(upd, lane + j * _LANES, best_i)
                if tail:
                    v = slab_v[par, r, pl.ds(vocab - _LANES, _LANES)]
                    upd = v > best_v
                    best_v = jnp.where(upd, v, best_v)
                    best_i = jnp.where(upd, lane + (vocab - _LANES), best_i)
                bv_buf[pl.ds(off + r * _LANES, _LANES)] = best_v
                bi_buf[pl.ds(off + r * _LANES, _LANES)] = best_i
                return 0

            lax.fori_loop(0, n, row_step, 0, unroll=False)
            pltpu.async_copy(
                bv_buf.at[pl.ds(off, row_w)],
                bv_hbm.at[pl.ds((base + s) * row_w, row_w)], osem)
            pltpu.async_copy(
                bi_buf.at[pl.ds(off, row_w)],
                bi_hbm.at[pl.ds((base + s) * row_w, row_w)], osem)
            return 0

        lax.fori_loop(0, bpw, slab_step, 0, unroll=False)
        for _ in range(2):
            pltpu.make_async_copy(
                bv_buf.at[pl.ds(0, row_w)],
                bv_hbm.at[pl.ds(base * row_w, row_w)], osem).wait()
            pltpu.make_async_copy(
                bi_buf.at[pl.ds(0, row_w)],
                bi_hbm.at[pl.ds(base * row_w, row_w)], osem).wait()

    mesh = plsc.VectorSubcoreMesh(core_axis_name="c", subcore_axis_name="s")
    return pl.kernel(
        _body,
        mesh=mesh,
        compiler_params=pltpu.CompilerParams(use_tc_tiling_on_sc=True),
        out_type=[
            jax.ShapeDtypeStruct((b * n * _LANES,), jnp.float32),
            jax.ShapeDtypeStruct((b * n * _LANES,), jnp.int32),
        ],
        scratch_types=[
            pltpu.VMEM((2, n, vocab), jnp.float32),
            pltpu.VMEM((2048,), jnp.float32),
            pltpu.VMEM((2048,), jnp.int32),
            pltpu.SemaphoreType.DMA,
            pltpu.SemaphoreType.DMA,
        ],
    )


def _make_fingather(rows, emb, vocab, n_workers):
    bpw = rows // n_workers
    n_groups = bpw // _LANES
    half = bpw // 2
    hchunks = half // _CHUNK

    def _body(bv_hbm, bi_hbm, table_hbm, out_hbm,
              bvv, biv, tok_v, rows_v, sem):
        wid = lax.axis_index("s") * 2 + lax.axis_index("c")
        lane = lax.iota(jnp.int32, _LANES)
        base = wid * bpw
        pltpu.sync_copy(bv_hbm.at[pl.ds(base * _LANES, bpw * _LANES)], bvv)
        pltpu.sync_copy(bi_hbm.at[pl.ds(base * _LANES, bpw * _LANES)], biv)

        def grp_step(g, _):
            acc = jnp.zeros((_LANES,), jnp.int32)
            for k in range(_LANES):
                o = (g * _LANES + k) * _LANES
                bv16 = bvv[pl.ds(o, _LANES)]
                bi16 = biv[pl.ds(o, _LANES)]
                ms = plsc.cummax(lax.rev(plsc.cummax(bv16), (0,)))
                sel = jnp.where(bv16 == ms, bi16, vocab)
                nm = plsc.cummax(lax.rev(plsc.cummax(-sel), (0,)))
                acc = jnp.where(lane == k, -nm, acc)
            tok_v[pl.ds(g * _LANES, _LANES)] = acc
            return 0

        lax.fori_loop(0, n_groups, grp_step, 0, unroll=False)

        for h in range(2):
            copies = [
                pltpu.async_copy(
                    table_hbm.at[tok_v.at[pl.ds(h * half + j * _CHUNK, _CHUNK)]],
                    rows_v.at[pl.ds(j * _CHUNK, _CHUNK)],
                    sem,
                )
                for j in range(hchunks)
            ]
            for cp in copies:
                cp.wait()
            pltpu.sync_copy(
                rows_v, out_hbm.at[pl.ds(base + h * half, half)])

    mesh = plsc.VectorSubcoreMesh(core_axis_name="c", subcore_axis_name="s")
    return pl.kernel(
        _body,
        mesh=mesh,
        compiler_params=pltpu.CompilerParams(
            use_tc_tiling_on_sc=False, needs_layout_passes=False),
        out_type=jax.ShapeDtypeStruct((rows, emb), jnp.float32),
        scratch_types=[
            pltpu.VMEM((bpw * _LANES,), jnp.float32),
            pltpu.VMEM((bpw * _LANES,), jnp.int32),
            pltpu.VMEM((bpw,), jnp.int32),
            pltpu.VMEM((bpw // 2, emb), jnp.float32),
            pltpu.SemaphoreType.DMA,
        ],
    )


def kernel(x, dictionary):
    b, n, vocab = x.shape
    emb = dictionary.shape[1]
    rows = b * n
    n_workers = 32

    bv, bi = _make_argmax_part1(b, n, vocab, n_workers)(x)
    out = _make_fingather(rows, emb, vocab, n_workers)(bv, bi, dictionary)
    return out.reshape(b, n, emb)


# speedup vs baseline: 1.0360x; 1.0360x over previous
import functools

import jax
import jax.numpy as jnp
from jax import lax
from jax.experimental import pallas as pl
from jax.experimental.pallas import tpu as pltpu
from jax.experimental.pallas import tpu_sc as plsc

_ROWS_PER_BLOCK = 1024
_CHUNK = 80


def _argmax_half(xb):
    vocab = xb.shape[-1]
    m = jnp.max(xb, axis=-1, keepdims=True)
    col = jax.lax.broadcasted_iota(jnp.int32, xb.shape, 1)
    return jnp.min(jnp.where(xb == m, col, vocab), axis=-1)


def _argmax_body(xa_ref, xb_ref, ta_ref, tb_ref):
    ta_ref[...] = _argmax_half(xa_ref[...])
    tb_ref[...] = _argmax_half(xb_ref[...])


def _compute_tokens(x2):
    rows, vocab = x2.shape
    grid = rows // (2 * _ROWS_PER_BLOCK)
    blk = (_ROWS_PER_BLOCK, vocab)
    return pl.pallas_call(
        _argmax_body,
        grid=(grid,),
        in_specs=[
            pl.BlockSpec(blk, lambda i: (i, 0)),
            pl.BlockSpec(blk, lambda i, g=grid: (i + g, 0)),
        ],
        out_specs=[
            pl.BlockSpec((_ROWS_PER_BLOCK,), lambda i: (i,)),
            pl.BlockSpec((_ROWS_PER_BLOCK,), lambda i: (i,)),
        ],
        out_shape=[
            jax.ShapeDtypeStruct((rows // 2,), jnp.int32),
            jax.ShapeDtypeStruct((rows // 2,), jnp.int32),
        ],
    )(x2, x2)


def _make_gather(rows, emb, n_workers, n_chunks):
    bpw = rows // n_workers
    half_workers = n_workers // 2

    def _gather_body(ta_hbm, tb_hbm, table_hbm, out_hbm, idx_v, rows_v, sem):
        wid = lax.axis_index("s") * 2 + lax.axis_index("c")
        @pl.when(wid < half_workers)
        def _():
            pltpu.sync_copy(ta_hbm.at[pl.ds(wid * bpw, bpw)], idx_v)

        @pl.when(wid >= half_workers)
        def _():
            pltpu.sync_copy(
                tb_hbm.at[pl.ds((wid - half_workers) * bpw, bpw)], idx_v)

        copies = [
            pltpu.async_copy(
                table_hbm.at[idx_v.at[pl.ds(j * _CHUNK, _CHUNK)]],
                rows_v.at[pl.ds(j * _CHUNK, _CHUNK)],
                sem,
            )
            for j in range(n_chunks)
        ]
        for cp in copies:
            cp.wait()
        pltpu.sync_copy(rows_v, out_hbm.at[pl.ds(wid * bpw, bpw)])

    mesh = plsc.VectorSubcoreMesh(core_axis_name="c", subcore_axis_name="s")
    return pl.kernel(
        _gather_body,
        mesh=mesh,
        compiler_params=pltpu.CompilerParams(use_tc_tiling_on_sc=False),
        out_type=jax.ShapeDtypeStruct((rows, emb), jnp.float32),
        scratch_types=[
            pltpu.VMEM((bpw,), jnp.int32),
            pltpu.VMEM((bpw, emb), jnp.float32),
            pltpu.SemaphoreType.DMA,
        ],
    )


def kernel(x, dictionary):
    b, n, vocab = x.shape
    emb = dictionary.shape[1]
    rows = b * n
    n_workers = 32
    n_chunks = rows // (n_workers * _CHUNK)

    ta, tb = _compute_tokens(x.reshape(rows, vocab))
    out = _make_gather(rows, emb, n_workers, n_chunks)(ta, tb, dictionary)
    return out.reshape(b, n, emb)
